# Initial kernel scaffold; baseline (speedup 1.0000x reference)
import functools

import jax
import jax.numpy as jnp
from jax import lax
from jax.experimental import pallas as pl
from jax.experimental.pallas import tpu as pltpu
from jax.experimental.pallas import tpu_sc as plsc

N = 10000
E = 320000
D = 128
H = 128
NB = 32

NC = 2
NS = 16
NW = NC * NS

CHUNK = 512
E_PAD = 327680
W_EDGES = E_PAD // NW
G_CHUNKS = W_EDGES // CHUNK
IDX_TOTAL = 2 * E_PAD
W_IDX = IDX_TOTAL // NW
GG_CHUNKS = W_IDX // CHUNK
JROWS = CHUNK // 128

NPT = N // NS

_mesh = plsc.VectorSubcoreMesh(
    core_axis_name="c", subcore_axis_name="s", num_cores=NC, num_subcores=NS)


@functools.partial(
    pl.kernel,
    out_type=(
        jax.ShapeDtypeStruct((IDX_TOTAL, D), jnp.float32),
        jax.ShapeDtypeStruct((IDX_TOTAL, 16), jnp.float32),
    ),
    mesh=_mesh,
    scratch_types=[
        pltpu.VMEM((JROWS, 128), jnp.int32),
        pltpu.VMEM((CHUNK, D), jnp.float32),
        pltpu.VMEM((CHUNK, 16), jnp.float32),
        pltpu.SemaphoreType.DMA,
        pltpu.SemaphoreType.DMA,
    ],
)
def _sc_gather(h_hbm, pos_hbm, idx_hbm, hg_out, pg_out,
               idx_v, hbuf, pbuf, semh, semp):
    wid = lax.axis_index("s") * NC + lax.axis_index("c")

    def chunk_body(g, carry):
        base = wid * W_IDX + g * CHUNK
        rbase = wid * (W_IDX // 128) + g * JROWS
        pltpu.sync_copy(idx_hbm.at[pl.ds(rbase, JROWS)], idx_v)
        copies = []
        for j in range(JROWS):
            copies.append(pltpu.async_copy(
                h_hbm.at[idx_v.at[j]], hbuf.at[pl.ds(j * 128, 128)], semh))
            copies.append(pltpu.async_copy(
                pos_hbm.at[idx_v.at[j]], pbuf.at[pl.ds(j * 128, 128)], semp))
        for c in copies:
            c.wait()
        pltpu.sync_copy(hbuf, hg_out.at[pl.ds(base, CHUNK)])
        pltpu.sync_copy(pbuf, pg_out.at[pl.ds(base, CHUNK)])
        return carry

    lax.fori_loop(0, GG_CHUNKS, chunk_body, 0)


BE = 2560
N_EBLK = E_PAD // BE


def _edge_mlp_body(hrow, hcol, prow, pcol, wm1a, wm1b, wm1c, bm1, wm2, bm2,
                   wc1, bc1, wc2, bc2, coeff, offs, msg_out, coord_out):
    i = pl.program_id(0)
    e0 = i * BE
    rows = lax.broadcasted_iota(jnp.int32, (BE, 1), 0) + e0
    mask = rows < E

    d16 = prow[...] - pcol[...]
    sq = jnp.sum(d16 * d16, axis=1, keepdims=True)
    dist = jnp.sqrt(sq + 1e-8)
    demb = jnp.exp(coeff[0, 0] * (dist - offs[...]) ** 2)

    x = (jnp.dot(hrow[...], wm1a[...], preferred_element_type=jnp.float32)
         + jnp.dot(hcol[...], wm1b[...], preferred_element_type=jnp.float32)
         + jnp.dot(demb, wm1c[...], preferred_element_type=jnp.float32)
         + bm1[...])
    x = x * jax.nn.sigmoid(x)
    m = jnp.dot(x, wm2[...], preferred_element_type=jnp.float32) + bm2[...]
    msg_out[...] = jnp.where(mask, m, 0.0)

    y = jnp.dot(m, wc1[...], preferred_element_type=jnp.float32) + bc1[...]
    y = y * jax.nn.sigmoid(y)
    mult = jnp.dot(y, wc2[...], preferred_element_type=jnp.float32) + bc2[...]
    coord_out[...] = jnp.where(mask, d16 * mult, 0.0)


def _run_edge_mlp(hg, pg, Wm1, bm1, Wm2, bm2, Wc1, bc1, Wc2, bc2, coeff, offs):
    full = lambda shape: pl.BlockSpec(shape, lambda i: tuple(0 for _ in shape))
    grid_spec = pl.GridSpec(
        grid=(N_EBLK,),
        in_specs=[
            pl.BlockSpec((BE, D), lambda i: (i, 0)),
            pl.BlockSpec((BE, D), lambda i: (i + N_EBLK, 0)),
            pl.BlockSpec((BE, 16), lambda i: (i, 0)),
            pl.BlockSpec((BE, 16), lambda i: (i + N_EBLK, 0)),
            full((D, H)), full((D, H)), full((NB, H)), full((1, H)),
            full((H, H)), full((1, H)),
            full((H, H)), full((1, H)), full((H, 1)), full((1, 1)),
            full((1, 1)), full((1, NB)),
        ],
        out_specs=[
            pl.BlockSpec((BE, H), lambda i: (i, 0)),
            pl.BlockSpec((BE, 16), lambda i: (i, 0)),
        ],
    )
    return pl.pallas_call(
        _edge_mlp_body,
        grid_spec=grid_spec,
        out_shape=[
            jax.ShapeDtypeStruct((E_PAD, H), jnp.float32),
            jax.ShapeDtypeStruct((E_PAD, 16), jnp.float32),
        ],
        compiler_params=pltpu.CompilerParams(
            dimension_semantics=("arbitrary",)),
    )(hg, hg, pg, pg,
      Wm1[:D], Wm1[D:2 * D], Wm1[2 * D:], bm1.reshape(1, H),
      Wm2, bm2.reshape(1, H), Wc1, bc1.reshape(1, H), Wc2,
      bc2.reshape(1, 1), coeff, offs.reshape(1, NB))


@functools.partial(
    pl.kernel,
    out_type=(
        jax.ShapeDtypeStruct((NC, N, H), jnp.float32),
        jax.ShapeDtypeStruct((NC, N, 16), jnp.float32),
    ),
    mesh=_mesh,
    scratch_types=[
        pltpu.VMEM_SHARED((N, H), jnp.float32),
        pltpu.VMEM_SHARED((N, 16), jnp.float32),
        pltpu.VMEM((JROWS, 128), jnp.int32),
        pltpu.VMEM((CHUNK, H), jnp.float32),
        pltpu.VMEM((CHUNK, 16), jnp.float32),
    ],
)
def _sc_scatter(row_hbm, msg_hbm, coord_hbm, zm_hbm, zc_hbm,
                accm_out, accc_out,
                accm_s, accc_s, idx_v, msg_v, coord_v):
    cid = lax.axis_index("c")
    sid = lax.axis_index("s")
    wid = sid * NC + cid
    r0 = sid * NPT

    pltpu.sync_copy(zm_hbm.at[pl.ds(r0, NPT)], accm_s.at[pl.ds(r0, NPT)])
    pltpu.sync_copy(zc_hbm.at[pl.ds(r0, NPT)], accc_s.at[pl.ds(r0, NPT)])
    plsc.subcore_barrier()

    def chunk_body(g, carry):
        base = wid * W_EDGES + g * CHUNK
        rbase = wid * (W_EDGES // 128) + g * JROWS
        pltpu.sync_copy(row_hbm.at[pl.ds(rbase, JROWS)], idx_v)
        pltpu.sync_copy(msg_hbm.at[pl.ds(base, CHUNK)], msg_v)
        pltpu.sync_copy(coord_hbm.at[pl.ds(base, CHUNK)], coord_v)
        for j in range(JROWS):
            pltpu.sync_copy(msg_v.at[pl.ds(j * 128, 128)],
                            accm_s.at[idx_v.at[j]], add=True)
            pltpu.sync_copy(coord_v.at[pl.ds(j * 128, 128)],
                            accc_s.at[idx_v.at[j]], add=True)
        return carry

    lax.fori_loop(0, G_CHUNKS, chunk_body, 0)

    plsc.subcore_barrier()
    pltpu.sync_copy(accm_s.at[pl.ds(r0, NPT)],
                    accm_out.at[cid].at[pl.ds(r0, NPT)])
    pltpu.sync_copy(accc_s.at[pl.ds(r0, NPT)],
                    accc_out.at[cid].at[pl.ds(r0, NPT)])


BN = 1000
N_NBLK = N // BN


def _node_mlp_body(hb, m0, m1, c0, c1, pb, wn1a, wn1b, bn1, wn2, bn2,
                   h_out, pos_out):
    aggr = m0[0] + m1[0]
    x = (jnp.dot(hb[...], wn1a[...], preferred_element_type=jnp.float32)
         + jnp.dot(aggr, wn1b[...], preferred_element_type=jnp.float32)
         + bn1[...])
    x = x * jax.nn.sigmoid(x)
    h_out[...] = (jnp.dot(x, wn2[...], preferred_element_type=jnp.float32)
                  + bn2[...] + hb[...])
    pos_out[...] = pb[...] + c0[0] + c1[0]


def _run_node_mlp(h, accm, accc, pos16, Wn1, bn1, Wn2, bn2):
    full = lambda shape: pl.BlockSpec(shape, lambda i: tuple(0 for _ in shape))
    grid_spec = pl.GridSpec(
        grid=(N_NBLK,),
        in_specs=[
            pl.BlockSpec((BN, D), lambda i: (i, 0)),
            pl.BlockSpec((1, BN, H), lambda i: (0, i, 0)),
            pl.BlockSpec((1, BN, H), lambda i: (1, i, 0)),
            pl.BlockSpec((1, BN, 16), lambda i: (0, i, 0)),
            pl.BlockSpec((1, BN, 16), lambda i: (1, i, 0)),
            pl.BlockSpec((BN, 16), lambda i: (i, 0)),
            full((D, H)), full((H, H)), full((1, H)),
            full((H, D)), full((1, D)),
        ],
        out_specs=[
            pl.BlockSpec((BN, D), lambda i: (i, 0)),
            pl.BlockSpec((BN, 16), lambda i: (i, 0)),
        ],
    )
    return pl.pallas_call(
        _node_mlp_body,
        grid_spec=grid_spec,
        out_shape=[
            jax.ShapeDtypeStruct((N, D), jnp.float32),
            jax.ShapeDtypeStruct((N, 16), jnp.float32),
        ],
        compiler_params=pltpu.CompilerParams(
            dimension_semantics=("arbitrary",)),
    )(h, accm, accm, accc, accc, pos16,
      Wn1[:D], Wn1[D:], bn1.reshape(1, H), Wn2, bn2.reshape(1, D))


def kernel(h, pos, edge_index, Wm1, bm1, Wm2, bm2, Wn1, bn1, Wn2, bn2,
           Wc1, bc1, Wc2, bc2, offset):
    row = edge_index[0]
    col = edge_index[1]
    padz = jnp.zeros((E_PAD - E,), jnp.int32)
    row_pad = jnp.concatenate([row, padz])
    idx_all = jnp.concatenate([row_pad, col, padz]).reshape(-1, 128)
    row2d = row_pad.reshape(-1, 128)

    pos16 = jnp.zeros((N, 16), jnp.float32).at[:, :3].set(pos)
    coeff = (-0.5 / (offset[1] - offset[0]) ** 2).reshape(1, 1)

    hg, pg = _sc_gather(h, pos16, idx_all)
    msg, coord = _run_edge_mlp(hg, pg, Wm1, bm1, Wm2, bm2, Wc1, bc1,
                               Wc2, bc2, coeff, offset)
    zm = jnp.zeros((N, H), jnp.float32)
    zc = jnp.zeros((N, 16), jnp.float32)
    accm, accc = _sc_scatter(row2d, msg, coord, zm, zc)
    h_out, pos_out16 = _run_node_mlp(h, accm, accc, pos16, Wn1, bn1, Wn2, bn2)
    return (h_out, pos_out16[:, :3])


# speedup vs baseline: 2.2702x; 2.2702x over previous
import functools

import jax
import jax.numpy as jnp
from jax import lax
from jax.experimental import pallas as pl
from jax.experimental.pallas import tpu as pltpu
from jax.experimental.pallas import tpu_sc as plsc

N = 10000
E = 320000
D = 128
H = 128
NB = 32

NC = 2
NS = 16
NW = NC * NS

HC = H // 2
CHUNK = 512
GRP = 1024
E_PAD = 327680
W_EDGES = E_PAD // NW
IDX_TOTAL = 2 * E_PAD
W_IDX = IDX_TOTAL // NW
G1_GRPS = W_IDX // GRP
G2_CHUNKS = W_IDX // CHUNK
SM_GRPS = (E_PAD // NS) // GRP
SC_GRPS = (E_PAD // NC // NS) // GRP

N_ACC = 10240
NPT = N_ACC // NS

_mesh = plsc.VectorSubcoreMesh(
    core_axis_name="c", subcore_axis_name="s", num_cores=NC, num_subcores=NS)


@functools.partial(
    pl.kernel,
    out_type=jax.ShapeDtypeStruct((IDX_TOTAL, D), jnp.float32),
    mesh=_mesh,
    scratch_types=[
        pltpu.VMEM((8, 128), jnp.int32),
        pltpu.VMEM((CHUNK, D), jnp.float32),
        pltpu.SemaphoreType.DMA,
    ],
)
def _sc_gather_h(h_hbm, idx_hbm, hg_out, idx_v, hbuf, semh):
    wid = lax.axis_index("s") * NC + lax.axis_index("c")

    def grp_body(g, carry):
        gbase = pl.multiple_of(wid * W_IDX + g * GRP, CHUNK)
        rbase = pl.multiple_of(wid * (W_IDX // 128) + g * 8, 8)
        pltpu.sync_copy(idx_hbm.at[pl.ds(rbase, 8)], idx_v)
        for half in range(2):
            copies = []
            for j in range(4):
                copies.append(pltpu.async_copy(
                    h_hbm.at[idx_v.at[half * 4 + j]],
                    hbuf.at[pl.ds(j * 128, 128)], semh))
            for c in copies:
                c.wait()
            pltpu.sync_copy(hbuf, hg_out.at[pl.ds(gbase + half * CHUNK, CHUNK)])
        return carry

    lax.fori_loop(0, G1_GRPS, grp_body, 0)


@functools.partial(
    pl.kernel,
    out_type=jax.ShapeDtypeStruct((IDX_TOTAL, 16), jnp.float32),
    mesh=_mesh,
    scratch_types=[
        pltpu.VMEM((4, 128), jnp.int32),
        pltpu.VMEM((CHUNK, 16), jnp.float32),
        pltpu.SemaphoreType.DMA,
    ],
    compiler_params=pltpu.CompilerParams(use_tc_tiling_on_sc=False),
)
def _sc_gather_pos(pos_hbm, idx_hbm, pg_out, idx_v, pbuf, semp):
    wid = lax.axis_index("s") * NC + lax.axis_index("c")

    def chunk_body(g, carry):
        base = pl.multiple_of(wid * W_IDX + g * CHUNK, CHUNK)
        rbase = pl.multiple_of(wid * (W_IDX // 128) + g * 4, 4)
        pltpu.sync_copy(idx_hbm.at[pl.ds(rbase, 4)], idx_v)
        copies = []
        for j in range(4):
            copies.append(pltpu.async_copy(
                pos_hbm.at[idx_v.at[j]], pbuf.at[pl.ds(j * 128, 128)], semp))
        for c in copies:
            c.wait()
        pltpu.sync_copy(pbuf, pg_out.at[pl.ds(base, CHUNK)])
        return carry

    lax.fori_loop(0, G2_CHUNKS, chunk_body, 0)


BE = 2560
N_EBLK = E_PAD // BE


def _edge_mlp_body(hrow, hcol, prow, pcol, wm1a, wm1b, wm1c, bm1, wm2, bm2,
                   wc1, bc1, wc2, bc2, coeff, offs,
                   mlo_out, mhi_out, coord_out):
    i = pl.program_id(0)
    rows = lax.broadcasted_iota(jnp.int32, (BE, 1), 0) + i * BE
    mask = rows < E

    d16 = prow[...] - pcol[...]
    sq = jnp.sum(d16 * d16, axis=1, keepdims=True)
    dist = jnp.sqrt(sq + 1e-8)
    demb = jnp.exp(coeff[0, 0] * (dist - offs[...]) ** 2)

    x = (jnp.dot(hrow[...], wm1a[...], preferred_element_type=jnp.float32)
         + jnp.dot(hcol[...], wm1b[...], preferred_element_type=jnp.float32)
         + jnp.dot(demb, wm1c[...], preferred_element_type=jnp.float32)
         + bm1[...])
    x = x * jax.nn.sigmoid(x)
    m = jnp.dot(x, wm2[...], preferred_element_type=jnp.float32) + bm2[...]
    m = jnp.where(mask, m, 0.0)
    mlo_out[...] = m[:, :HC]
    mhi_out[...] = m[:, HC:]

    y = jnp.dot(m, wc1[...], preferred_element_type=jnp.float32) + bc1[...]
    y = y * jax.nn.sigmoid(y)
    mult = jnp.dot(y, wc2[...], preferred_element_type=jnp.float32) + bc2[...]
    coord_out[...] = jnp.where(mask, d16 * mult, 0.0)


def _run_edge_mlp(hg, pg, Wm1, bm1, Wm2, bm2, Wc1, bc1, Wc2, bc2, coeff, offs):
    full = lambda shape: pl.BlockSpec(shape, lambda i: tuple(0 for _ in shape))
    grid_spec = pl.GridSpec(
        grid=(N_EBLK,),
        in_specs=[
            pl.BlockSpec((BE, D), lambda i: (i, 0)),
            pl.BlockSpec((BE, D), lambda i: (i + N_EBLK, 0)),
            pl.BlockSpec((BE, 16), lambda i: (i, 0)),
            pl.BlockSpec((BE, 16), lambda i: (i + N_EBLK, 0)),
            full((D, H)), full((D, H)), full((NB, H)), full((1, H)),
            full((H, H)), full((1, H)),
            full((H, H)), full((1, H)), full((H, 1)), full((1, 1)),
            full((1, 1)), full((1, NB)),
        ],
        out_specs=[
            pl.BlockSpec((BE, HC), lambda i: (i, 0)),
            pl.BlockSpec((BE, HC), lambda i: (i, 0)),
            pl.BlockSpec((BE, 16), lambda i: (i, 0)),
        ],
    )
    return pl.pallas_call(
        _edge_mlp_body,
        grid_spec=grid_spec,
        out_shape=[
            jax.ShapeDtypeStruct((E_PAD, HC), jnp.float32),
            jax.ShapeDtypeStruct((E_PAD, HC), jnp.float32),
            jax.ShapeDtypeStruct((E_PAD, 16), jnp.float32),
        ],
        compiler_params=pltpu.CompilerParams(
            dimension_semantics=("arbitrary",)),
    )(hg, hg, pg, pg,
      Wm1[:D], Wm1[D:2 * D], Wm1[2 * D:], bm1.reshape(1, H),
      Wm2, bm2.reshape(1, H), Wc1, bc1.reshape(1, H), Wc2,
      bc2.reshape(1, 1), coeff, offs.reshape(1, NB))


@functools.partial(
    pl.kernel,
    out_type=(
        jax.ShapeDtypeStruct((NC, N_ACC, HC), jnp.float32),
        jax.ShapeDtypeStruct((NC, N_ACC, 16), jnp.float32),
    ),
    mesh=_mesh,
    scratch_types=[
        pltpu.VMEM_SHARED((N_ACC, HC), jnp.float32),
        pltpu.VMEM_SHARED((N_ACC, 16), jnp.float32),
        pltpu.VMEM((8, 128), jnp.int32),
        pltpu.VMEM((CHUNK, HC), jnp.float32),
        pltpu.VMEM((GRP, 16), jnp.float32),
    ],
    compiler_params=pltpu.CompilerParams(use_tc_tiling_on_sc=False),
)
def _sc_scatter(row_hbm, mlo_hbm, mhi_hbm, coord_hbm, zm_hbm, zc_hbm,
                accm_out, accc_out,
                accm_s, accc_s, idx_v, msg_v, coord_v):
    cid = lax.axis_index("c")
    sid = lax.axis_index("s")
    r0 = pl.multiple_of(sid * NPT, NPT)

    pltpu.sync_copy(zm_hbm.at[pl.ds(r0, NPT)], accm_s.at[pl.ds(r0, NPT)])
    pltpu.sync_copy(zc_hbm.at[pl.ds(r0, NPT)], accc_s.at[pl.ds(r0, NPT)])
    plsc.subcore_barrier()

    def msg_body(g, carry):
        gbase = pl.multiple_of(sid * (E_PAD // NS) + g * GRP, GRP)
        pltpu.sync_copy(row_hbm.at[pl.ds(pl.multiple_of(gbase // 128, 8), 8)], idx_v)
        for half in range(2):
            base = pl.multiple_of(gbase + half * CHUNK, CHUNK)

            @pl.when(cid == 0)
            def _():
                pltpu.sync_copy(mlo_hbm.at[pl.ds(base, CHUNK)], msg_v)

            @pl.when(cid == 1)
            def _():
                pltpu.sync_copy(mhi_hbm.at[pl.ds(base, CHUNK)], msg_v)

            for j in range(4):
                pltpu.sync_copy(msg_v.at[pl.ds(j * 128, 128)],
                                accm_s.at[idx_v.at[half * 4 + j]], add=True)
        return carry

    lax.fori_loop(0, SM_GRPS, msg_body, 0)

    def coord_body(g, carry):
        gbase = pl.multiple_of(
            cid * (E_PAD // NC) + sid * (E_PAD // NC // NS) + g * GRP, GRP)
        pltpu.sync_copy(row_hbm.at[pl.ds(pl.multiple_of(gbase // 128, 8), 8)], idx_v)
        pltpu.sync_copy(coord_hbm.at[pl.ds(gbase, GRP)], coord_v)
        for j in range(8):
            pltpu.sync_copy(coord_v.at[pl.ds(j * 128, 128)],
                            accc_s.at[idx_v.at[j]], add=True)
        return carry

    lax.fori_loop(0, SC_GRPS, coord_body, 0)

    plsc.subcore_barrier()
    pltpu.sync_copy(accm_s.at[pl.ds(r0, NPT)],
                    accm_out.at[cid].at[pl.ds(r0, NPT)])
    pltpu.sync_copy(accc_s.at[pl.ds(r0, NPT)],
                    accc_out.at[cid].at[pl.ds(r0, NPT)])


BN = 1000
N_NBLK = N // BN


def _node_mlp_body(hb, mlo, mhi, c0, c1, pb, wn1a, wn1lo, wn1hi, bn1,
                   wn2, bn2, h_out, pos_out):
    x = (jnp.dot(hb[...], wn1a[...], preferred_element_type=jnp.float32)
         + jnp.dot(mlo[0], wn1lo[...], preferred_element_type=jnp.float32)
         + jnp.dot(mhi[0], wn1hi[...], preferred_element_type=jnp.float32)
         + bn1[...])
    x = x * jax.nn.sigmoid(x)
    h_out[...] = (jnp.dot(x, wn2[...], preferred_element_type=jnp.float32)
                  + bn2[...] + hb[...])
    pos_out[...] = pb[...] + c0[0] + c1[0]


def _run_node_mlp(h, accm, accc, pos16, Wn1, bn1, Wn2, bn2):
    full = lambda shape: pl.BlockSpec(shape, lambda i: tuple(0 for _ in shape))
    grid_spec = pl.GridSpec(
        grid=(N_NBLK,),
        in_specs=[
            pl.BlockSpec((BN, D), lambda i: (i, 0)),
            pl.BlockSpec((1, BN, HC), lambda i: (0, i, 0)),
            pl.BlockSpec((1, BN, HC), lambda i: (1, i, 0)),
            pl.BlockSpec((1, BN, 16), lambda i: (0, i, 0)),
            pl.BlockSpec((1, BN, 16), lambda i: (1, i, 0)),
            pl.BlockSpec((BN, 16), lambda i: (i, 0)),
            full((D, H)), full((HC, H)), full((HC, H)), full((1, H)),
            full((H, D)), full((1, D)),
        ],
        out_specs=[
            pl.BlockSpec((BN, D), lambda i: (i, 0)),
            pl.BlockSpec((BN, 16), lambda i: (i, 0)),
        ],
    )
    return pl.pallas_call(
        _node_mlp_body,
        grid_spec=grid_spec,
        out_shape=[
            jax.ShapeDtypeStruct((N, D), jnp.float32),
            jax.ShapeDtypeStruct((N, 16), jnp.float32),
        ],
        compiler_params=pltpu.CompilerParams(
            dimension_semantics=("arbitrary",)),
    )(h, accm, accm, accc, accc, pos16,
      Wn1[:D], Wn1[D:D + HC], Wn1[D + HC:], bn1.reshape(1, H),
      Wn2, bn2.reshape(1, D))


def kernel(h, pos, edge_index, Wm1, bm1, Wm2, bm2, Wn1, bn1, Wn2, bn2,
           Wc1, bc1, Wc2, bc2, offset):
    row = edge_index[0]
    col = edge_index[1]
    padz = jnp.zeros((E_PAD - E,), jnp.int32)
    row_pad = jnp.concatenate([row, padz])
    idx_all = jnp.concatenate([row_pad, col, padz]).reshape(-1, 128)
    row2d = row_pad.reshape(-1, 128)

    pos16 = jnp.zeros((N, 16), jnp.float32).at[:, :3].set(pos)
    coeff = (-0.5 / (offset[1] - offset[0]) ** 2).reshape(1, 1)

    hg = _sc_gather_h(h, idx_all)
    pg = _sc_gather_pos(pos16, idx_all)
    mlo, mhi, coord = _run_edge_mlp(hg, pg, Wm1, bm1, Wm2, bm2, Wc1, bc1,
                                    Wc2, bc2, coeff, offset)
    zm = jnp.zeros((N_ACC, HC), jnp.float32)
    zc = jnp.zeros((N_ACC, 16), jnp.float32)
    accm, accc = _sc_scatter(row2d, mlo, mhi, coord, zm, zc)
    h_out, pos_out16 = _run_node_mlp(h, accm, accc, pos16, Wn1, bn1, Wn2, bn2)
    return (h_out, pos_out16[:, :3])


# speedup vs baseline: 4.1616x; 1.8331x over previous
import functools

import jax
import jax.numpy as jnp
from jax import lax
from jax.experimental import pallas as pl
from jax.experimental.pallas import tpu as pltpu
from jax.experimental.pallas import tpu_sc as plsc

N = 10000
E = 320000
D = 128
H = 128
NB = 32

NC = 2
NS = 16
NW = NC * NS

HC = H // 2
TW = HC + 16
CHUNK = 512
GRP = 1024
E_PAD = 327680
W_EDGES = E_PAD // NW
IDX_TOTAL = 2 * E_PAD
W_IDX = IDX_TOTAL // NW
G1_GRPS = W_IDX // GRP
G2_CHUNKS = W_IDX // CHUNK
SM_GRPS = (E_PAD // NS) // GRP
SC_GRPS = (E_PAD // NC // NS) // GRP

N_ACC = 10240
NPT = N_ACC // NS

_mesh = plsc.VectorSubcoreMesh(
    core_axis_name="c", subcore_axis_name="s", num_cores=NC, num_subcores=NS)


BP = 1000
N_PBLK = N // BP


def _pack_bf16_pair(x):
    xb = jax.lax.bitcast_convert_type(x, jnp.int32)
    rnd = jnp.int32(0x8000)
    lo = ((xb[:, :HC] + rnd) >> 16) & jnp.int32(0xFFFF)
    hi = (xb[:, HC:] + rnd) & jnp.int32(-65536)
    return lo | hi


def _proj_body(hb, wa, wb, b1, a_out, b_out):
    a = (jnp.dot(hb[...], wa[...], preferred_element_type=jnp.float32)
         + b1[...])
    b = jnp.dot(hb[...], wb[...], preferred_element_type=jnp.float32)
    a_out[...] = _pack_bf16_pair(a)
    b_out[...] = _pack_bf16_pair(b)


def _run_proj(h, Wm1, bm1):
    full = lambda shape: pl.BlockSpec(shape, lambda i: tuple(0 for _ in shape))
    grid_spec = pl.GridSpec(
        grid=(N_PBLK,),
        in_specs=[
            pl.BlockSpec((BP, D), lambda i: (i, 0)),
            full((D, H)), full((D, H)), full((1, H)),
        ],
        out_specs=[
            pl.BlockSpec((BP, HC), lambda i: (i, 0)),
            pl.BlockSpec((BP, HC), lambda i: (i, 0)),
        ],
    )
    return pl.pallas_call(
        _proj_body,
        grid_spec=grid_spec,
        out_shape=[
            jax.ShapeDtypeStruct((N, HC), jnp.int32),
            jax.ShapeDtypeStruct((N, HC), jnp.int32),
        ],
        compiler_params=pltpu.CompilerParams(
            dimension_semantics=("arbitrary",)),
    )(h, Wm1[:D], Wm1[D:2 * D], bm1.reshape(1, H))


CG = 256
NCH = W_EDGES // CG
CGR = CG // 128


@functools.partial(
    pl.kernel,
    out_type=(
        jax.ShapeDtypeStruct((E_PAD, H), jnp.int32),
        jax.ShapeDtypeStruct((E_PAD, 16), jnp.float32),
        jax.ShapeDtypeStruct((E_PAD, 16), jnp.float32),
    ),
    mesh=_mesh,
    scratch_types=[
        pltpu.VMEM((2, CGR, 128), jnp.int32),
        pltpu.VMEM((2, CGR, 128), jnp.int32),
        pltpu.VMEM((2, CG, HC), jnp.int32),
        pltpu.VMEM((2, CG, HC), jnp.int32),
        pltpu.VMEM((2, CG, 16), jnp.float32),
        pltpu.VMEM((2, CG, 16), jnp.float32),
        pltpu.SemaphoreType.DMA,
        pltpu.SemaphoreType.DMA,
        pltpu.SemaphoreType.DMA,
        pltpu.SemaphoreType.DMA,
    ],
    compiler_params=pltpu.CompilerParams(use_tc_tiling_on_sc=False),
)
def _sc_gather(a_hbm, b_hbm, p_hbm, idxr_hbm, idxc_hbm,
               agb_out, pgr_out, pgc_out,
               idxr_v, idxc_v, bufa, bufb, bufpr, bufpc,
               semi, semg, semp, semw):
    wid = lax.axis_index("s") * NC + lax.axis_index("c")
    rbase0 = wid * (W_EDGES // 128)
    ebase0 = wid * W_EDGES

    def start_idx(g):
        b = g % 2
        r = pl.multiple_of(rbase0 + g * CGR, CGR)
        return (pltpu.async_copy(idxr_hbm.at[pl.ds(r, CGR)], idxr_v.at[b], semi),
                pltpu.async_copy(idxc_hbm.at[pl.ds(r, CGR)], idxc_v.at[b], semi))

    def start_gathers(g):
        b = g % 2
        ds_ = []
        for j in range(CGR):
            sl = pl.ds(j * 128, 128)
            ds_.append(pltpu.async_copy(
                a_hbm.at[idxr_v.at[b].at[j]], bufa.at[b].at[sl], semg))
            ds_.append(pltpu.async_copy(
                b_hbm.at[idxc_v.at[b].at[j]], bufb.at[b].at[sl], semg))
            ds_.append(pltpu.async_copy(
                p_hbm.at[idxr_v.at[b].at[j]], bufpr.at[b].at[sl], semp))
            ds_.append(pltpu.async_copy(
                p_hbm.at[idxc_v.at[b].at[j]], bufpc.at[b].at[sl], semp))
        return ds_

    def start_wb(g):
        b = g % 2
        eb = pl.multiple_of(ebase0 + g * CG, CG)
        return (pltpu.async_copy(
                    bufa.at[b], agb_out.at[pl.ds(eb, CG), pl.ds(0, HC)], semw),
                pltpu.async_copy(
                    bufb.at[b], agb_out.at[pl.ds(eb, CG), pl.ds(HC, HC)], semw),
                pltpu.async_copy(bufpr.at[b], pgr_out.at[pl.ds(eb, CG)], semw),
                pltpu.async_copy(bufpc.at[b], pgc_out.at[pl.ds(eb, CG)], semw))

    idx_d = {0: start_idx(0)}
    wb_d = {}
    for g in range(NCH):
        for d_ in idx_d.pop(g):
            d_.wait()
        if g + 1 < NCH:
            idx_d[g + 1] = start_idx(g + 1)
        if g - 2 in wb_d:
            for d_ in wb_d.pop(g - 2):
                d_.wait()
        gds = start_gathers(g)
        for d_ in gds:
            d_.wait()
        wb_d[g] = start_wb(g)
    for g in sorted(wb_d):
        for d_ in wb_d[g]:
            d_.wait()


BE = 5120
N_EBLK = E_PAD // BE


def _unpack_bf16_pair(p):
    lo = jax.lax.bitcast_convert_type(p << 16, jnp.float32)
    hi = jax.lax.bitcast_convert_type(p & jnp.int32(-65536), jnp.float32)
    return lo, hi


def _edge_mlp_body(agb, prow, pcol, wm1c, wm2, bm2,
                   wc1, bc1, wc2, bc2, coeff, offs,
                   msg_out, coord_out):
    i = pl.program_id(0)
    rows = lax.broadcasted_iota(jnp.int32, (BE, 1), 0) + i * BE
    mask = rows < E

    d16 = prow[...] - pcol[...]
    sq = jnp.sum(d16 * d16, axis=1, keepdims=True)
    dist = jnp.sqrt(sq + 1e-8)
    demb = jnp.exp(coeff[0, 0] * (dist - offs[...]) ** 2)

    ap = agb[...]
    alo, ahi = _unpack_bf16_pair(ap[:, :HC])
    blo, bhi = _unpack_bf16_pair(ap[:, HC:])
    x = (jnp.concatenate([alo + blo, ahi + bhi], axis=1)
         + jnp.dot(demb, wm1c[...], preferred_element_type=jnp.float32))
    x = x * jax.nn.sigmoid(x)
    m = jnp.dot(x, wm2[...], preferred_element_type=jnp.float32) + bm2[...]
    m = jnp.where(mask, m, 0.0)
    msg_out[...] = m

    y = jnp.dot(m, wc1[...], preferred_element_type=jnp.float32) + bc1[...]
    y = y * jax.nn.sigmoid(y)
    mult = jnp.dot(y, wc2[...], preferred_element_type=jnp.float32) + bc2[...]
    coord_out[...] = jnp.where(mask, d16 * mult, 0.0)


def _run_edge_mlp(agb, pgr, pgc, Wm1, Wm2, bm2, Wc1, bc1, Wc2, bc2,
                  coeff, offs):
    full = lambda shape: pl.BlockSpec(shape, lambda i: tuple(0 for _ in shape))
    grid_spec = pl.GridSpec(
        grid=(N_EBLK,),
        in_specs=[
            pl.BlockSpec((BE, D), lambda i: (i, 0)),
            pl.BlockSpec((BE, 16), lambda i: (i, 0)),
            pl.BlockSpec((BE, 16), lambda i: (i, 0)),
            full((NB, H)),
            full((H, H)), full((1, H)),
            full((H, H)), full((1, H)), full((H, 1)), full((1, 1)),
            full((1, 1)), full((1, NB)),
        ],
        out_specs=[
            pl.BlockSpec((BE, H), lambda i: (i, 0)),
            pl.BlockSpec((BE, 16), lambda i: (i, 0)),
        ],
    )
    return pl.pallas_call(
        _edge_mlp_body,
        grid_spec=grid_spec,
        out_shape=[
            jax.ShapeDtypeStruct((E_PAD, H), jnp.float32),
            jax.ShapeDtypeStruct((E_PAD, 16), jnp.float32),
        ],
        compiler_params=pltpu.CompilerParams(
            dimension_semantics=("arbitrary",)),
    )(agb, pgr, pgc,
      Wm1[2 * D:],
      Wm2, bm2.reshape(1, H), Wc1, bc1.reshape(1, H), Wc2,
      bc2.reshape(1, 1), coeff, offs.reshape(1, NB))


MCH = 256
NMCH = (E_PAD // NS) // MCH
CCH = 512
NCCH = (E_PAD // NC // NS) // CCH


@functools.partial(
    pl.kernel,
    out_type=(
        jax.ShapeDtypeStruct((NC, N_ACC, HC), jnp.float32),
        jax.ShapeDtypeStruct((NC, N_ACC, 16), jnp.float32),
    ),
    mesh=_mesh,
    scratch_types=[
        pltpu.VMEM_SHARED((N_ACC, HC), jnp.float32),
        pltpu.VMEM_SHARED((N_ACC, 16), jnp.float32),
        pltpu.VMEM((2, 8, 128), jnp.int32),
        pltpu.VMEM((2, MCH, HC), jnp.float32),
        pltpu.VMEM((2, CCH, 16), jnp.float32),
        pltpu.SemaphoreType.DMA,
        pltpu.SemaphoreType.DMA,
        pltpu.SemaphoreType.DMA,
    ],
    compiler_params=pltpu.CompilerParams(use_tc_tiling_on_sc=False),
)
def _sc_scatter(row_hbm, msg_hbm, coord_hbm, zm_hbm, zc_hbm,
                accm_out, accc_out,
                accm_s, accc_s, idx_v, msg_v, coord_v,
                semi, semm, sems):
    cid = lax.axis_index("c")
    sid = lax.axis_index("s")
    r0 = pl.multiple_of(sid * NPT, NPT)

    pltpu.sync_copy(zm_hbm.at[pl.ds(r0, NPT)], accm_s.at[pl.ds(r0, NPT)])
    pltpu.sync_copy(zc_hbm.at[pl.ds(r0, NPT)], accc_s.at[pl.ds(r0, NPT)])
    plsc.subcore_barrier()

    ebase0 = sid * (E_PAD // NS)
    rbase0 = sid * (E_PAD // NS // 128)
    lane0 = pl.multiple_of(cid * HC, HC)
    idx_per = 1024 // MCH

    def m_idx(k):
        r = pl.multiple_of(rbase0 + k * 8, 8)
        return pltpu.async_copy(row_hbm.at[pl.ds(r, 8)], idx_v.at[k % 2], semi)

    def m_load(g):
        eb = pl.multiple_of(ebase0 + g * MCH, MCH)
        return pltpu.async_copy(
            msg_hbm.at[pl.ds(eb, MCH), pl.ds(lane0, HC)], msg_v.at[g % 2], semm)

    def m_scat(g):
        k = (g // idx_per) % 2
        j0 = (g % idx_per) * (MCH // 128)
        out = []
        for j in range(MCH // 128):
            out.append(pltpu.async_copy(
                msg_v.at[g % 2].at[pl.ds(j * 128, 128)],
                accm_s.at[idx_v.at[k].at[j0 + j]], sems, add=True))
        return out

    idx_d = {0: m_idx(0)}
    load_d = {0: m_load(0)}
    scat_d = {}
    for g in range(NMCH):
        if g - 1 in scat_d:
            for d_ in scat_d.pop(g - 1):
                d_.wait()
        if g % idx_per == 0:
            idx_d.pop(g // idx_per).wait()
            nk = g // idx_per + 1
            if nk * idx_per < NMCH:
                idx_d[nk] = m_idx(nk)
        if g + 1 < NMCH:
            load_d[g + 1] = m_load(g + 1)
        load_d.pop(g).wait()
        scat_d[g] = m_scat(g)
    for g in sorted(scat_d):
        for d_ in scat_d[g]:
            d_.wait()

    cbase0 = cid * (E_PAD // NC) + sid * (E_PAD // NC // NS)
    crbase0 = cbase0 // 128
    c_idx_per = 1024 // CCH

    def c_idx(k):
        r = pl.multiple_of(crbase0 + k * 8, 8)
        return pltpu.async_copy(row_hbm.at[pl.ds(r, 8)], idx_v.at[k % 2], semi)

    def c_load(g):
        eb = pl.multiple_of(cbase0 + g * CCH, CCH)
        return pltpu.async_copy(
            coord_hbm.at[pl.ds(eb, CCH)], coord_v.at[g % 2], semm)

    def c_scat(g):
        k = (g // c_idx_per) % 2
        j0 = (g % c_idx_per) * (CCH // 128)
        out = []
        for j in range(CCH // 128):
            out.append(pltpu.async_copy(
                coord_v.at[g % 2].at[pl.ds(j * 128, 128)],
                accc_s.at[idx_v.at[k].at[j0 + j]], sems, add=True))
        return out

    idx_d = {0: c_idx(0)}
    load_d = {0: c_load(0)}
    scat_d = {}
    for g in range(NCCH):
        if g - 1 in scat_d:
            for d_ in scat_d.pop(g - 1):
                d_.wait()
        if g % c_idx_per == 0:
            idx_d.pop(g // c_idx_per).wait()
            nk = g // c_idx_per + 1
            if nk * c_idx_per < NCCH:
                idx_d[nk] = c_idx(nk)
        if g + 1 < NCCH:
            load_d[g + 1] = c_load(g + 1)
        load_d.pop(g).wait()
        scat_d[g] = c_scat(g)
    for g in sorted(scat_d):
        for d_ in scat_d[g]:
            d_.wait()

    plsc.subcore_barrier()
    pltpu.sync_copy(accm_s.at[pl.ds(r0, NPT)],
                    accm_out.at[cid].at[pl.ds(r0, NPT)])
    pltpu.sync_copy(accc_s.at[pl.ds(r0, NPT)],
                    accc_out.at[cid].at[pl.ds(r0, NPT)])


BN = 1000
N_NBLK = N // BN


def _node_mlp_body(hb, mlo, mhi, c0, c1, pb, wn1a, wn1lo, wn1hi, bn1,
                   wn2, bn2, h_out, pos_out):
    x = (jnp.dot(hb[...], wn1a[...], preferred_element_type=jnp.float32)
         + jnp.dot(mlo[0], wn1lo[...], preferred_element_type=jnp.float32)
         + jnp.dot(mhi[0], wn1hi[...], preferred_element_type=jnp.float32)
         + bn1[...])
    x = x * jax.nn.sigmoid(x)
    h_out[...] = (jnp.dot(x, wn2[...], preferred_element_type=jnp.float32)
                  + bn2[...] + hb[...])
    pos_out[...] = pb[...] + c0[0] + c1[0]


def _run_node_mlp(h, accm, accc, pos16, Wn1, bn1, Wn2, bn2):
    full = lambda shape: pl.BlockSpec(shape, lambda i: tuple(0 for _ in shape))
    grid_spec = pl.GridSpec(
        grid=(N_NBLK,),
        in_specs=[
            pl.BlockSpec((BN, D), lambda i: (i, 0)),
            pl.BlockSpec((1, BN, HC), lambda i: (0, i, 0)),
            pl.BlockSpec((1, BN, HC), lambda i: (1, i, 0)),
            pl.BlockSpec((1, BN, 16), lambda i: (0, i, 0)),
            pl.BlockSpec((1, BN, 16), lambda i: (1, i, 0)),
            pl.BlockSpec((BN, 16), lambda i: (i, 0)),
            full((D, H)), full((HC, H)), full((HC, H)), full((1, H)),
            full((H, D)), full((1, D)),
        ],
        out_specs=[
            pl.BlockSpec((BN, D), lambda i: (i, 0)),
            pl.BlockSpec((BN, 16), lambda i: (i, 0)),
        ],
    )
    return pl.pallas_call(
        _node_mlp_body,
        grid_spec=grid_spec,
        out_shape=[
            jax.ShapeDtypeStruct((N, D), jnp.float32),
            jax.ShapeDtypeStruct((N, 16), jnp.float32),
        ],
        compiler_params=pltpu.CompilerParams(
            dimension_semantics=("arbitrary",)),
    )(h, accm, accm, accc, accc, pos16,
      Wn1[:D], Wn1[D:D + HC], Wn1[D + HC:], bn1.reshape(1, H),
      Wn2, bn2.reshape(1, D))


def kernel(h, pos, edge_index, Wm1, bm1, Wm2, bm2, Wn1, bn1, Wn2, bn2,
           Wc1, bc1, Wc2, bc2, offset):
    row = edge_index[0]
    col = edge_index[1]
    padz = jnp.zeros((E_PAD - E,), jnp.int32)
    row_pad = jnp.concatenate([row, padz])
    col_pad = jnp.concatenate([col, padz])
    row2d = row_pad.reshape(-1, 128)
    col2d = col_pad.reshape(-1, 128)

    pos16 = jnp.zeros((N, 16), jnp.float32).at[:, :3].set(pos)
    coeff = (-0.5 / (offset[1] - offset[0]) ** 2).reshape(1, 1)

    a_n, b_n = _run_proj(h, Wm1, bm1)
    agb, pgr, pgc = _sc_gather(a_n, b_n, pos16, row2d, col2d)
    msg, coord = _run_edge_mlp(agb, pgr, pgc, Wm1, Wm2, bm2, Wc1,
                               bc1, Wc2, bc2, coeff, offset)
    zm = jnp.zeros((N_ACC, HC), jnp.float32)
    zc = jnp.zeros((N_ACC, 16), jnp.float32)
    accm, accc = _sc_scatter(row2d, msg, coord, zm, zc)
    h_out, pos_out16 = _run_node_mlp(h, accm, accc, pos16, Wn1, bn1, Wn2, bn2)
    return (h_out, pos_out16[:, :3])
